# baseline (speedup 1.0000x reference)
import functools

import jax
import jax.numpy as jnp
from jax import lax
from jax.experimental import pallas as pl
from jax.experimental.pallas import tpu as pltpu
from jax.experimental.pallas import tpu_sc as plsc

_N = 1024 * 1024
_K = 734002
_ONE_BITS = 0x3F800000

_NBIN = 4096
_NB = 16 * _NBIN
_WLO = 0.6951
_SCALE = 400000.0
_INV = 1.0 / _SCALE

_NW = 32
_EPW = (8 * _N) // _NW
_CHUNK = 8192
_NCHUNK = _EPW // _CHUNK


def _sc_hist_body(adj_hbm, hist_hbm, buf_a, buf_b, hist_v, sem_a, sem_b):
    wid = lax.axis_index("s") * 2 + lax.axis_index("c")
    base = wid * _EPW

    def zero_body(i, _):
        hist_v[pl.ds(i * 16, 16)] = jnp.zeros((16,), jnp.int32)
        return 0

    lax.fori_loop(0, _NB // 16, zero_body, 0)

    bufs = (buf_a, buf_b)
    sems = (sem_a, sem_b)

    def copy_in(ci, slot):
        return pltpu.make_async_copy(
            adj_hbm.at[pl.ds(base + ci * _CHUNK, _CHUNK)], bufs[slot], sems[slot])

    copy_in(0, 0).start()
    copy_in(1, 1).start()

    ones = jnp.ones((16,), jnp.int32)
    lane_off = lax.iota(jnp.int32, 16) * _NBIN

    def process(buf):
        def body(j, _):
            v = buf[pl.ds(j * 16, 16)]
            t = (v - _WLO) * _SCALE
            bi = t.astype(jnp.int32) + 1
            bi = jnp.maximum(bi, 0)
            bi = jnp.minimum(bi, _NBIN - 1)
            plsc.addupdate_scatter(hist_v, [bi + lane_off], ones)
            return 0

        lax.fori_loop(0, _CHUNK // 16, body, 0)

    def outer(io, _):
        for slot in range(2):
            ci = io * 2 + slot
            copy_in(ci, slot).wait()
            process(bufs[slot])

            @pl.when(ci + 2 < _NCHUNK)
            def _():
                copy_in(ci + 2, slot).start()

        return 0

    lax.fori_loop(0, _NCHUNK // 2, outer, 0)
    pltpu.sync_copy(hist_v, hist_hbm.at[wid])


@functools.cache
def _get_sc_hist():
    @functools.partial(
        pl.kernel,
        mesh=plsc.VectorSubcoreMesh(core_axis_name="c", subcore_axis_name="s"),
        compiler_params=pltpu.CompilerParams(needs_layout_passes=False),
        out_type=jax.ShapeDtypeStruct((_NW, _NB), jnp.int32),
        scratch_types=[
            pltpu.VMEM((_CHUNK,), jnp.float32),
            pltpu.VMEM((_CHUNK,), jnp.float32),
            pltpu.VMEM((_NB,), jnp.int32),
            pltpu.SemaphoreType.DMA,
            pltpu.SemaphoreType.DMA,
        ],
    )
    def _sc_hist(adj_hbm, hist_hbm, buf_a, buf_b, hist_v, sem_a, sem_b):
        _sc_hist_body(adj_hbm, hist_hbm, buf_a, buf_b, hist_v, sem_a, sem_b)

    return _sc_hist


def _tc_gcn_kernel(adj_ref, x_ref, w1_ref, b1_ref, w2_ref, b2_ref, hist_ref,
                   out_ref):
    b = pl.program_id(0)

    hm = jnp.sum(hist_ref[0].astype(jnp.float32), axis=0)
    ci = lax.broadcasted_iota(jnp.int32, (128, 128), 0)
    cj = lax.broadcasted_iota(jnp.int32, (128, 128), 1)
    u_incl = (ci <= cj).astype(jnp.float32)
    ri = lax.broadcasted_iota(jnp.int32, (32, 32), 0)
    rj = lax.broadcasted_iota(jnp.int32, (32, 32), 1)
    l_strict = (rj < ri).astype(jnp.float32)
    crow = jnp.dot(hm, u_incl, preferred_element_type=jnp.float32)
    row_tot = crow[:, 127:128]
    pref = jnp.dot(l_strict, row_tot, preferred_element_type=jnp.float32)
    cum = crow + pref
    j1 = jnp.sum((cum < float(_K + 1)).astype(jnp.int32))

    valid = jnp.logical_and(j1 >= 2, j1 <= _NBIN - 3)
    lo_e = jnp.float32(_WLO) + (j1 - 2).astype(jnp.float32) * jnp.float32(_INV)
    hi_e = jnp.float32(_WLO) + (j1 + 1).astype(jnp.float32) * jnp.float32(_INV)
    lo_bits0 = lax.bitcast_convert_type(jnp.maximum(lo_e, 0.0), jnp.int32)
    hi_bits0 = lax.bitcast_convert_type(jnp.minimum(hi_e, 1.0), jnp.int32)
    lo_cand = jnp.where(valid, lo_bits0, jnp.int32(0))
    hi_cand = jnp.where(valid, hi_bits0, jnp.int32(_ONE_BITS))

    def count_le(t):
        return jnp.sum((adj_ref[0] <= t).astype(jnp.int32))

    c_lo = count_le(lax.bitcast_convert_type(lo_cand, jnp.float32))
    c_hi = count_le(lax.bitcast_convert_type(hi_cand, jnp.float32))
    good = jnp.logical_and(c_lo <= _K, c_hi >= _K + 1)
    lo_init = jnp.where(good, lo_cand, jnp.int32(0))
    hi_init = jnp.where(good, hi_cand, jnp.int32(_ONE_BITS))

    def cond_fn(carry):
        lo, hi = carry
        return lo < hi

    def body_fn(carry):
        lo, hi = carry
        mid = (lo + hi) // 2
        t = lax.bitcast_convert_type(mid, jnp.float32)
        pred = count_le(t) >= _K + 1
        return jnp.where(pred, lo, mid + 1), jnp.where(pred, mid, hi)

    lo, _ = lax.while_loop(cond_fn, body_fn, (lo_init, hi_init))
    a_k = lax.bitcast_convert_type(lo, jnp.float32)

    adj = adj_ref[0]
    c_le = jnp.sum((adj <= a_k).astype(jnp.int32))
    a_next = jnp.min(jnp.where(adj > a_k, adj, 2.0))
    a_k1 = jnp.where(c_le >= _K + 2, a_k, a_next)
    thresh = 0.5 * a_k + 0.5 * a_k1

    mask = (adj_ref[0] > thresh).astype(jnp.float32)
    h0 = jnp.dot(x_ref[0], w1_ref[...], preferred_element_type=jnp.float32)
    h0 = h0 + b1_ref[...]
    h1 = jnp.maximum(jnp.dot(mask, h0, preferred_element_type=jnp.float32), 0.0)
    h2 = jnp.dot(h1, w2_ref[...], preferred_element_type=jnp.float32)
    h2 = h2 + b2_ref[...]
    h2 = jnp.maximum(jnp.dot(mask, h2, preferred_element_type=jnp.float32), 0.0)
    out_ref[pl.ds(b, 1), :] = jnp.mean(h2, axis=0, keepdims=True)


def _tc_gcn(x, adj, W1, b1, W2, b2, hist4):
    bsz = adj.shape[0]
    return pl.pallas_call(
        _tc_gcn_kernel,
        grid=(bsz,),
        in_specs=[
            pl.BlockSpec((1, 1024, 1024), lambda b: (b, 0, 0)),
            pl.BlockSpec((1, 1024, 128), lambda b: (b, 0, 0)),
            pl.BlockSpec((128, 128), lambda b: (0, 0)),
            pl.BlockSpec((1, 128), lambda b: (0, 0)),
            pl.BlockSpec((128, 128), lambda b: (0, 0)),
            pl.BlockSpec((1, 128), lambda b: (0, 0)),
            pl.BlockSpec((1, 64, 32, 128), lambda b: (b, 0, 0, 0)),
        ],
        out_specs=pl.BlockSpec((bsz, 128), lambda b: (0, 0)),
        out_shape=jax.ShapeDtypeStruct((bsz, 128), jnp.float32),
    )(adj, x, W1, b1.reshape(1, 128), W2, b2.reshape(1, 128), hist4)


@jax.jit
def kernel(x, adj, W1, b1, W2, b2):
    bsz = adj.shape[0]
    adj_flat = adj.reshape(bsz * 1024 * 1024)
    hist = _get_sc_hist()(adj_flat)
    hist4 = hist.reshape(bsz, 64, 32, 128)
    return _tc_gcn(x, adj, W1, b1, W2, b2, hist4)


# speedup vs baseline: 1.4453x; 1.4453x over previous
import functools

import jax
import jax.numpy as jnp
from jax import lax
from jax.experimental import pallas as pl
from jax.experimental.pallas import tpu as pltpu
from jax.experimental.pallas import tpu_sc as plsc

_N = 1024 * 1024
_K = 734002
_ONE_BITS = 0x3F800000

_NBIN = 4096
_NB = 16 * _NBIN
_WLO = 0.6951
_SCALE = 400000.0
_INV = 1.0 / _SCALE

_NW = 32
_EPW = (8 * _N) // _NW
_CHUNK = 8192
_NCHUNK = _EPW // _CHUNK


def _sc_hist_body(adj_hbm, hist_hbm, buf_a, buf_b, hist_v, sem_a, sem_b):
    wid = lax.axis_index("s") * 2 + lax.axis_index("c")
    base = wid * _EPW

    def zero_body(i, _):
        hist_v[pl.ds(i * 16, 16)] = jnp.zeros((16,), jnp.int32)
        return 0

    lax.fori_loop(0, _NB // 16, zero_body, 0)

    bufs = (buf_a, buf_b)
    sems = (sem_a, sem_b)

    def copy_in(ci, slot):
        return pltpu.make_async_copy(
            adj_hbm.at[pl.ds(base + ci * _CHUNK, _CHUNK)], bufs[slot], sems[slot])

    copy_in(0, 0).start()
    copy_in(1, 1).start()

    ones = jnp.ones((16,), jnp.int32)
    lane_off = lax.iota(jnp.int32, 16) * _NBIN
    off = jnp.float32(1.0 - _WLO * _SCALE)

    def process(buf):
        def body(j, _):
            for k in range(4):
                v = buf[pl.ds(j * 64 + k * 16, 16)]
                t = v * _SCALE + off
                bi = t.astype(jnp.int32)
                bi = jnp.maximum(bi, 0)
                bi = jnp.minimum(bi, _NBIN - 1)
                plsc.addupdate_scatter(hist_v, [bi + lane_off], ones)
            return 0

        lax.fori_loop(0, _CHUNK // 64, body, 0)

    def outer(io, _):
        for slot in range(2):
            ci = io * 2 + slot
            copy_in(ci, slot).wait()
            process(bufs[slot])

            @pl.when(ci + 2 < _NCHUNK)
            def _():
                copy_in(ci + 2, slot).start()

        return 0

    lax.fori_loop(0, _NCHUNK // 2, outer, 0)
    pltpu.sync_copy(hist_v, hist_hbm.at[wid])


@functools.cache
def _get_sc_hist():
    @functools.partial(
        pl.kernel,
        mesh=plsc.VectorSubcoreMesh(core_axis_name="c", subcore_axis_name="s"),
        compiler_params=pltpu.CompilerParams(needs_layout_passes=False),
        out_type=jax.ShapeDtypeStruct((_NW, _NB), jnp.int32),
        scratch_types=[
            pltpu.VMEM((_CHUNK,), jnp.float32),
            pltpu.VMEM((_CHUNK,), jnp.float32),
            pltpu.VMEM((_NB,), jnp.int32),
            pltpu.SemaphoreType.DMA,
            pltpu.SemaphoreType.DMA,
        ],
    )
    def _sc_hist(adj_hbm, hist_hbm, buf_a, buf_b, hist_v, sem_a, sem_b):
        _sc_hist_body(adj_hbm, hist_hbm, buf_a, buf_b, hist_v, sem_a, sem_b)

    return _sc_hist


def _tc_gcn_kernel(adj_ref, x_ref, w1_ref, b1_ref, w2_ref, b2_ref, hist_ref,
                   out_ref):
    b = pl.program_id(0)

    hm = jnp.sum(hist_ref[0].astype(jnp.float32), axis=0)
    bin_idx = (lax.broadcasted_iota(jnp.int32, (32, 128), 0) * 128
               + lax.broadcasted_iota(jnp.int32, (32, 128), 1))

    def loc_body(_, carry):
        blo, bhi = carry
        bmid = (blo + bhi) // 2
        c = jnp.sum(jnp.where(bin_idx <= bmid, hm, 0.0))
        pred = c >= float(_K + 1)
        return jnp.where(pred, blo, bmid + 1), jnp.where(pred, bmid, bhi)

    j1, _ = lax.fori_loop(0, 12, loc_body, (jnp.int32(0), jnp.int32(_NBIN - 1)))

    valid = jnp.logical_and(j1 >= 2, j1 <= _NBIN - 3)
    lo_e = jnp.float32(_WLO) + (j1 - 2).astype(jnp.float32) * jnp.float32(_INV)
    hi_e = jnp.float32(_WLO) + (j1 + 1).astype(jnp.float32) * jnp.float32(_INV)
    lo_bits0 = lax.bitcast_convert_type(jnp.maximum(lo_e, 0.0), jnp.int32)
    hi_bits0 = lax.bitcast_convert_type(jnp.minimum(hi_e, 1.0), jnp.int32)
    lo_cand = jnp.where(valid, lo_bits0, jnp.int32(0))
    hi_cand = jnp.where(valid, hi_bits0, jnp.int32(_ONE_BITS))

    def count_le(t):
        return jnp.sum((adj_ref[0] <= t).astype(jnp.int32))

    c_lo = count_le(lax.bitcast_convert_type(lo_cand, jnp.float32))
    c_hi = count_le(lax.bitcast_convert_type(hi_cand, jnp.float32))
    good = jnp.logical_and(c_lo <= _K, c_hi >= _K + 1)
    lo_init = jnp.where(good, lo_cand, jnp.int32(0))
    hi_init = jnp.where(good, hi_cand, jnp.int32(_ONE_BITS))

    def cond_fn(carry):
        lo, hi = carry
        return lo < hi

    def body_fn(carry):
        lo, hi = carry
        mid = (lo + hi) // 2
        t = lax.bitcast_convert_type(mid, jnp.float32)
        pred = count_le(t) >= _K + 1
        return jnp.where(pred, lo, mid + 1), jnp.where(pred, mid, hi)

    lo, _ = lax.while_loop(cond_fn, body_fn, (lo_init, hi_init))
    a_k = lax.bitcast_convert_type(lo, jnp.float32)

    adj = adj_ref[0]
    c_le = jnp.sum((adj <= a_k).astype(jnp.int32))
    a_next = jnp.min(jnp.where(adj > a_k, adj, 2.0))
    a_k1 = jnp.where(c_le >= _K + 2, a_k, a_next)
    thresh = 0.5 * a_k + 0.5 * a_k1

    mask = (adj_ref[0] > thresh).astype(jnp.float32)
    h0 = jnp.dot(x_ref[0], w1_ref[...], preferred_element_type=jnp.float32)
    h0 = h0 + b1_ref[...]
    h1 = jnp.maximum(jnp.dot(mask, h0, preferred_element_type=jnp.float32), 0.0)
    h2 = jnp.dot(h1, w2_ref[...], preferred_element_type=jnp.float32)
    h2 = h2 + b2_ref[...]
    h2 = jnp.maximum(jnp.dot(mask, h2, preferred_element_type=jnp.float32), 0.0)
    out_ref[pl.ds(b, 1), :] = jnp.mean(h2, axis=0, keepdims=True)


def _tc_gcn(x, adj, W1, b1, W2, b2, hist4):
    bsz = adj.shape[0]
    return pl.pallas_call(
        _tc_gcn_kernel,
        grid=(bsz,),
        in_specs=[
            pl.BlockSpec((1, 1024, 1024), lambda b: (b, 0, 0)),
            pl.BlockSpec((1, 1024, 128), lambda b: (b, 0, 0)),
            pl.BlockSpec((128, 128), lambda b: (0, 0)),
            pl.BlockSpec((1, 128), lambda b: (0, 0)),
            pl.BlockSpec((128, 128), lambda b: (0, 0)),
            pl.BlockSpec((1, 128), lambda b: (0, 0)),
            pl.BlockSpec((1, 64, 32, 128), lambda b: (b, 0, 0, 0)),
        ],
        out_specs=pl.BlockSpec((bsz, 128), lambda b: (0, 0)),
        out_shape=jax.ShapeDtypeStruct((bsz, 128), jnp.float32),
    )(adj, x, W1, b1.reshape(1, 128), W2, b2.reshape(1, 128), hist4)


@jax.jit
def kernel(x, adj, W1, b1, W2, b2):
    bsz = adj.shape[0]
    adj_flat = adj.reshape(bsz * 1024 * 1024)
    hist = _get_sc_hist()(adj_flat)
    hist4 = hist.reshape(bsz, 64, 32, 128)
    return _tc_gcn(x, adj, W1, b1, W2, b2, hist4)
